# Initial kernel scaffold; baseline (speedup 1.0000x reference)
import jax
import jax.numpy as jnp
from jax.experimental import pallas as pl


def kernel(users, movies, user_emb, movie_emb, fc_w, fc_b):
    raise NotImplementedError("write your pallas kernel here")


# speedup vs baseline: 6.0228x; 6.0228x over previous
import functools

import jax
import jax.numpy as jnp
from jax import lax
from jax.experimental import pallas as pl
from jax.experimental.pallas import tpu as pltpu
from jax.experimental.pallas import tpu_sc as plsc

EMBED = 128
BATCH = 16384
CHUNK = 128
NC, NS = 2, 16
NW = NC * NS
GROUPS = BATCH // CHUNK
G_PER_W = GROUPS // NW

_sc_mesh = plsc.VectorSubcoreMesh(core_axis_name="c", subcore_axis_name="s")


def _gather_body(users_hbm, movies_hbm, uemb_hbm, memb_hbm, u_out, m_out,
                 idx_v, rows_v, sem):
    wid = lax.axis_index("s") * NC + lax.axis_index("c")
    for g in range(G_PER_W):
        grow = wid * G_PER_W + g
        pltpu.sync_copy(users_hbm.at[grow], idx_v)
        pltpu.async_copy(uemb_hbm.at[idx_v], rows_v, sem).wait()
        pltpu.sync_copy(rows_v, u_out.at[pl.ds(grow * CHUNK, CHUNK)])
        pltpu.sync_copy(movies_hbm.at[grow], idx_v)
        pltpu.async_copy(memb_hbm.at[idx_v], rows_v, sem).wait()
        pltpu.sync_copy(rows_v, m_out.at[pl.ds(grow * CHUNK, CHUNK)])


_gather = pl.kernel(
    _gather_body,
    out_type=(
        jax.ShapeDtypeStruct((BATCH, EMBED), jnp.float32),
        jax.ShapeDtypeStruct((BATCH, EMBED), jnp.float32),
    ),
    mesh=_sc_mesh,
    scratch_types=[
        pltpu.VMEM((CHUNK,), jnp.int32),
        pltpu.VMEM((CHUNK, EMBED), jnp.float32),
        pltpu.SemaphoreType.DMA,
    ],
)


def _mm_body(u_ref, m_ref, wu_ref, wm_ref, b_ref, o_ref):
    acc = jnp.dot(u_ref[...], wu_ref[...], preferred_element_type=jnp.float32)
    acc = acc + jnp.dot(m_ref[...], wm_ref[...],
                        preferred_element_type=jnp.float32)
    o_ref[...] = acc + b_ref[...]


BM = 2048


def _matmul(u_rows, m_rows, wu, wm, b2):
    n_out = wu.shape[1]
    return pl.pallas_call(
        _mm_body,
        grid=(BATCH // BM,),
        in_specs=[
            pl.BlockSpec((BM, EMBED), lambda i: (i, 0)),
            pl.BlockSpec((BM, EMBED), lambda i: (i, 0)),
            pl.BlockSpec((EMBED, n_out), lambda i: (0, 0)),
            pl.BlockSpec((EMBED, n_out), lambda i: (0, 0)),
            pl.BlockSpec((1, n_out), lambda i: (0, 0)),
        ],
        out_specs=pl.BlockSpec((BM, n_out), lambda i: (i, 0)),
        out_shape=jax.ShapeDtypeStruct((BATCH, n_out), jnp.float32),
    )(u_rows, m_rows, wu, wm, b2)


def kernel(users, movies, user_emb, movie_emb, fc_w, fc_b):
    u_rows, m_rows = _gather(users.reshape(GROUPS, CHUNK),
                             movies.reshape(GROUPS, CHUNK),
                             user_emb, movie_emb)
    wu = fc_w[:, :EMBED].T
    wm = fc_w[:, EMBED:].T
    return _matmul(u_rows, m_rows, wu, wm, fc_b.reshape(1, -1))


# speedup vs baseline: 7.0606x; 1.1723x over previous
import functools

import jax
import jax.numpy as jnp
from jax import lax
from jax.experimental import pallas as pl
from jax.experimental.pallas import tpu as pltpu
from jax.experimental.pallas import tpu_sc as plsc

EMBED = 128
BATCH = 16384
CHUNK = 128
NC, NS = 2, 16
NW = NC * NS
GROUPS = BATCH // CHUNK
G_PER_W = GROUPS // NW

_sc_mesh = plsc.VectorSubcoreMesh(core_axis_name="c", subcore_axis_name="s")


NB = 4


def _gather_body(users_hbm, movies_hbm, uemb_hbm, memb_hbm, u_out, m_out,
                 idx_u, idx_m, rows, *sems):
    gsems, wsems = sems[:NB], sems[NB:]
    wid = lax.axis_index("s") * NC + lax.axis_index("c")
    gbase = wid * G_PER_W
    pltpu.sync_copy(users_hbm.at[pl.ds(gbase, G_PER_W)], idx_u)
    pltpu.sync_copy(movies_hbm.at[pl.ds(gbase, G_PER_W)], idx_m)
    chunks = ([(idx_u, uemb_hbm, u_out, g) for g in range(G_PER_W)]
              + [(idx_m, memb_hbm, m_out, g) for g in range(G_PER_W)])
    n = len(chunks)
    gdesc, wdesc = [None] * n, [None] * n

    def issue_write(j):
        _, _, out, gj = chunks[j]
        bj = j % NB
        gdesc[j].wait()
        wdesc[j] = pltpu.async_copy(
            rows.at[bj], out.at[pl.ds((gbase + gj) * CHUNK, CHUNK)], wsems[bj])

    for c in range(n):
        b = c % NB
        if c >= NB:
            wdesc[c - NB].wait()
        ix, tab, _, g = chunks[c]
        gdesc[c] = pltpu.async_copy(tab.at[ix.at[g]], rows.at[b], gsems[b])
        if c - (NB - 1) >= 0:
            issue_write(c - (NB - 1))
    for j in range(n - (NB - 1), n):
        issue_write(j)
    for j in range(n - NB, n):
        wdesc[j].wait()


_gather = pl.kernel(
    _gather_body,
    out_type=(
        jax.ShapeDtypeStruct((BATCH, EMBED), jnp.float32),
        jax.ShapeDtypeStruct((BATCH, EMBED), jnp.float32),
    ),
    mesh=_sc_mesh,
    scratch_types=(
        [pltpu.VMEM((G_PER_W, CHUNK), jnp.int32),
         pltpu.VMEM((G_PER_W, CHUNK), jnp.int32),
         pltpu.VMEM((NB, CHUNK, EMBED), jnp.float32)]
        + [pltpu.SemaphoreType.DMA] * (2 * NB)
    ),
)


def _mm_body(u_ref, m_ref, wu_ref, wm_ref, b_ref, o_ref):
    acc = jnp.dot(u_ref[...], wu_ref[...], preferred_element_type=jnp.float32)
    acc = acc + jnp.dot(m_ref[...], wm_ref[...],
                        preferred_element_type=jnp.float32)
    o_ref[...] = acc + b_ref[...]


BM = 2048


def _matmul(u_rows, m_rows, wu, wm, b2):
    n_out = wu.shape[1]
    return pl.pallas_call(
        _mm_body,
        grid=(BATCH // BM,),
        in_specs=[
            pl.BlockSpec((BM, EMBED), lambda i: (i, 0)),
            pl.BlockSpec((BM, EMBED), lambda i: (i, 0)),
            pl.BlockSpec((EMBED, n_out), lambda i: (0, 0)),
            pl.BlockSpec((EMBED, n_out), lambda i: (0, 0)),
            pl.BlockSpec((1, n_out), lambda i: (0, 0)),
        ],
        out_specs=pl.BlockSpec((BM, n_out), lambda i: (i, 0)),
        out_shape=jax.ShapeDtypeStruct((BATCH, n_out), jnp.float32),
    )(u_rows, m_rows, wu, wm, b2)


def kernel(users, movies, user_emb, movie_emb, fc_w, fc_b):
    u_rows, m_rows = _gather(users.reshape(GROUPS, CHUNK),
                             movies.reshape(GROUPS, CHUNK),
                             user_emb, movie_emb)
    wu = fc_w[:, :EMBED].T
    wm = fc_w[:, EMBED:].T
    return _matmul(u_rows, m_rows, wu, wm, fc_b.reshape(1, -1))


# speedup vs baseline: 8.4507x; 1.1969x over previous
import functools

import jax
import jax.numpy as jnp
from jax import lax
from jax.experimental import pallas as pl
from jax.experimental.pallas import tpu as pltpu
from jax.experimental.pallas import tpu_sc as plsc

EMBED = 128
BATCH = 16384
CHUNK = 128
NC, NS = 2, 16
NW = NC * NS
GROUPS = BATCH // CHUNK
G_PER_W = GROUPS // NW

_sc_mesh = plsc.VectorSubcoreMesh(core_axis_name="c", subcore_axis_name="s")


NB = 4


def _gather_body(users_hbm, movies_hbm, uemb_hbm, memb_hbm, u_out, m_out,
                 idx_u, idx_m, rows, *sems):
    gsems, wsems = sems[:NB], sems[NB:]
    wid = lax.axis_index("s") * NC + lax.axis_index("c")
    gbase = wid * G_PER_W
    pltpu.sync_copy(users_hbm.at[pl.ds(gbase, G_PER_W)], idx_u)
    pltpu.sync_copy(movies_hbm.at[pl.ds(gbase, G_PER_W)], idx_m)
    chunks = ([(idx_u, uemb_hbm, u_out, g) for g in range(G_PER_W)]
              + [(idx_m, memb_hbm, m_out, g) for g in range(G_PER_W)])
    n = len(chunks)
    gdesc, wdesc = [None] * n, [None] * n

    def issue_write(j):
        _, _, out, gj = chunks[j]
        bj = j % NB
        gdesc[j].wait()
        wdesc[j] = pltpu.async_copy(
            rows.at[bj], out.at[pl.ds((gbase + gj) * CHUNK, CHUNK)], wsems[bj])

    for c in range(n):
        b = c % NB
        if c >= NB:
            wdesc[c - NB].wait()
        ix, tab, _, g = chunks[c]
        gdesc[c] = pltpu.async_copy(tab.at[ix.at[g]], rows.at[b], gsems[b])
        if c - (NB - 1) >= 0:
            issue_write(c - (NB - 1))
    for j in range(n - (NB - 1), n):
        issue_write(j)
    for j in range(n - NB, n):
        wdesc[j].wait()


_gather = pl.kernel(
    _gather_body,
    out_type=(
        jax.ShapeDtypeStruct((BATCH, EMBED), jnp.float32),
        jax.ShapeDtypeStruct((BATCH, EMBED), jnp.float32),
    ),
    mesh=_sc_mesh,
    scratch_types=(
        [pltpu.VMEM((G_PER_W, CHUNK), jnp.int32),
         pltpu.VMEM((G_PER_W, CHUNK), jnp.int32),
         pltpu.VMEM((NB, CHUNK, EMBED), jnp.float32)]
        + [pltpu.SemaphoreType.DMA] * (2 * NB)
    ),
)


def _mm_body(u_ref, m_ref, wu_ref, wm_ref, b_ref, o_ref):
    dn = (((1,), (1,)), ((), ()))
    acc = lax.dot_general(wu_ref[...], u_ref[...], dn,
                          preferred_element_type=jnp.float32)
    acc = acc + lax.dot_general(wm_ref[...], m_ref[...], dn,
                                preferred_element_type=jnp.float32)
    o_ref[...] = acc + b_ref[...]


BM = 2048


def _matmul(u_rows, m_rows, wu, wm, b2):
    n_out = wu.shape[0]
    return pl.pallas_call(
        _mm_body,
        grid=(BATCH // BM,),
        in_specs=[
            pl.BlockSpec((BM, EMBED), lambda i: (i, 0)),
            pl.BlockSpec((BM, EMBED), lambda i: (i, 0)),
            pl.BlockSpec((n_out, EMBED), lambda i: (0, 0)),
            pl.BlockSpec((n_out, EMBED), lambda i: (0, 0)),
            pl.BlockSpec((n_out, 1), lambda i: (0, 0)),
        ],
        out_specs=pl.BlockSpec((n_out, BM), lambda i: (0, i)),
        out_shape=jax.ShapeDtypeStruct((n_out, BATCH), jnp.float32),
    )(u_rows, m_rows, wu, wm, b2)


def kernel(users, movies, user_emb, movie_emb, fc_w, fc_b):
    u_rows, m_rows = _gather(users.reshape(GROUPS, CHUNK),
                             movies.reshape(GROUPS, CHUNK),
                             user_emb, movie_emb)
    wu = fc_w[:, :EMBED]
    wm = fc_w[:, EMBED:]
    out_t = _matmul(u_rows, m_rows, wu, wm, fc_b.reshape(-1, 1))
    return out_t.T
